# Initial kernel scaffold; baseline (speedup 1.0000x reference)
import jax
import jax.numpy as jnp
from jax.experimental import pallas as pl


def kernel(indices, table):
    raise NotImplementedError("write your pallas kernel here")


# speedup vs baseline: 1.0225x; 1.0225x over previous
import functools

import jax
import jax.numpy as jnp
from jax import lax
from jax.experimental import pallas as pl
from jax.experimental.pallas import tpu as pltpu
from jax.experimental.pallas import tpu_sc as plsc

EMBED_DIM = 32
NUM_CORES = 2
NUM_SUBCORES = 16
NUM_WORKERS = NUM_CORES * NUM_SUBCORES
GROUP = 128


def _make_sc_gather(bflat):
    per_w = bflat // NUM_WORKERS
    ngroups = per_w // GROUP
    mesh = plsc.VectorSubcoreMesh(core_axis_name="c", subcore_axis_name="s")

    @functools.partial(
        pl.kernel,
        mesh=mesh,
        out_type=jax.ShapeDtypeStruct((bflat, EMBED_DIM), jnp.float32),
        scratch_types=[
            pltpu.VMEM((ngroups, GROUP), jnp.int32),
            pltpu.VMEM((GROUP, EMBED_DIM), jnp.float32),
            pltpu.SemaphoreType.DMA,
        ],
        compiler_params=pltpu.CompilerParams(use_tc_tiling_on_sc=False),
    )
    def k(table_hbm, idx_hbm, out_hbm, idx_v, rows_v, gsem):
        cid = lax.axis_index("c")
        sid = lax.axis_index("s")
        wid = sid * NUM_CORES + cid
        base = wid * per_w
        pltpu.sync_copy(idx_hbm.at[wid], idx_v)

        def body(g, carry):
            pltpu.async_copy(table_hbm.at[idx_v.at[g]], rows_v, gsem).wait()
            pltpu.sync_copy(rows_v, out_hbm.at[pl.ds(base + g * GROUP, GROUP)])
            return carry

        lax.fori_loop(0, ngroups, body, 0)

    return k


def kernel(indices, table):
    batch, hist = indices.shape
    bflat = batch * hist
    idx3 = indices.reshape(NUM_WORKERS, (bflat // NUM_WORKERS) // GROUP, GROUP)
    out_flat = _make_sc_gather(bflat)(table, idx3)
    return out_flat.reshape(batch, hist, EMBED_DIM)


# speedup vs baseline: 1.1100x; 1.0856x over previous
import functools

import jax
import jax.numpy as jnp
from jax import lax
from jax.experimental import pallas as pl
from jax.experimental.pallas import tpu as pltpu
from jax.experimental.pallas import tpu_sc as plsc

EMBED_DIM = 32
NUM_CORES = 2
NUM_SUBCORES = 16
NUM_WORKERS = NUM_CORES * NUM_SUBCORES
GROUP = 128
NBUF = 8


def _make_sc_gather(bflat):
    per_w = bflat // NUM_WORKERS
    ngroups = per_w // GROUP
    nsteps = ngroups // NBUF
    mesh = plsc.VectorSubcoreMesh(core_axis_name="c", subcore_axis_name="s")

    @functools.partial(
        pl.kernel,
        mesh=mesh,
        out_type=jax.ShapeDtypeStruct((bflat, EMBED_DIM), jnp.float32),
        scratch_types=[
            pltpu.VMEM((ngroups, GROUP), jnp.int32),
            pltpu.VMEM((NBUF, GROUP, EMBED_DIM), jnp.float32),
            pltpu.SemaphoreType.DMA((NBUF,)),
            pltpu.SemaphoreType.DMA((NBUF,)),
        ],
        compiler_params=pltpu.CompilerParams(use_tc_tiling_on_sc=False),
    )
    def k(table_hbm, idx_hbm, out_hbm, idx_v, rows_v, gsem, wsem):
        cid = lax.axis_index("c")
        sid = lax.axis_index("s")
        wid = sid * NUM_CORES + cid
        base = wid * per_w
        pltpu.sync_copy(idx_hbm.at[wid], idx_v)

        def gather_start(b, g):
            pltpu.make_async_copy(
                table_hbm.at[idx_v.at[g]], rows_v.at[b], gsem.at[b]
            ).start()

        def gather_wait(b, g):
            pltpu.make_async_copy(
                table_hbm.at[idx_v.at[g]], rows_v.at[b], gsem.at[b]
            ).wait()

        def write_start(b, g):
            pltpu.make_async_copy(
                rows_v.at[b], out_hbm.at[pl.ds(base + g * GROUP, GROUP)], wsem.at[b]
            ).start()

        def write_wait(b, g):
            pltpu.make_async_copy(
                rows_v.at[b], out_hbm.at[pl.ds(base + g * GROUP, GROUP)], wsem.at[b]
            ).wait()

        half = NBUF // 2

        for g in range(half):
            gather_start(g, g)

        def body(step, carry):
            g0 = step * NBUF
            for b in range(NBUF):
                g = g0 + b
                gather_wait(b, g)
                write_start(b, g)
                bq = (b + half) % NBUF
                gq = g + half
                gp = gq - NBUF

                @pl.when(gp >= 0)
                def _():
                    write_wait(bq, gp)

                @pl.when(gq < ngroups)
                def _():
                    gather_start(bq, gq)

            return carry

        lax.fori_loop(0, nsteps, body, 0)

        for g in range(ngroups - half, ngroups):
            write_wait(g % NBUF, g)

    return k


def kernel(indices, table):
    batch, hist = indices.shape
    bflat = batch * hist
    idx3 = indices.reshape(NUM_WORKERS, (bflat // NUM_WORKERS) // GROUP, GROUP)
    out_flat = _make_sc_gather(bflat)(table, idx3)
    return out_flat.reshape(batch, hist, EMBED_DIM)


# speedup vs baseline: 1.4545x; 1.3104x over previous
import functools

import jax
import jax.numpy as jnp
from jax import lax
from jax.experimental import pallas as pl
from jax.experimental.pallas import tpu as pltpu
from jax.experimental.pallas import tpu_sc as plsc

NUM_CORES = 2
NUM_SUBCORES = 16
NUM_WORKERS = NUM_CORES * NUM_SUBCORES
GROUP = 128
NBUF = 4


def _make_sc_gather(batch, hist, embed, vocab):
    cblocks = batch // GROUP
    cb_per_w = cblocks // NUM_WORKERS
    b_per_w = batch // NUM_WORKERS
    ngroups = hist * cb_per_w
    dr = embed // 8
    mesh = plsc.VectorSubcoreMesh(core_axis_name="c", subcore_axis_name="s")

    @functools.partial(
        pl.kernel,
        mesh=mesh,
        out_type=jax.ShapeDtypeStruct((hist, dr, cblocks, 8, GROUP), jnp.float32),
        scratch_types=[
            pltpu.VMEM((b_per_w, hist), jnp.int32),
            pltpu.VMEM((hist, cb_per_w, GROUP), jnp.int32),
            pltpu.VMEM((NBUF, GROUP, embed), jnp.float32),
            pltpu.VMEM((NBUF, dr, 8, GROUP), jnp.float32),
            pltpu.SemaphoreType.DMA((NBUF,)),
            pltpu.SemaphoreType.DMA((NBUF,)),
        ],
        compiler_params=pltpu.CompilerParams(
            use_tc_tiling_on_sc=False, needs_layout_passes=False
        ),
    )
    def k(table_hbm, idx_hbm, out_hbm, idx_v, idx_t, rows_v, tbuf, gsem, wsem):
        cid = lax.axis_index("c")
        sid = lax.axis_index("s")
        wid = sid * NUM_CORES + cid
        b0 = wid * b_per_w
        c0 = wid * cb_per_w
        lanes = lax.iota(jnp.int32, 16)

        pltpu.sync_copy(idx_hbm.at[pl.ds(b0, b_per_w)], idx_v)

        def tr_idx(h, carry):
            for cl in range(cb_per_w):
                for l0 in range(0, GROUP, 16):
                    vec = plsc.load_gather(
                        idx_v,
                        [cl * GROUP + l0 + lanes, jnp.full((16,), h, jnp.int32)],
                    )
                    idx_t[h, cl, pl.ds(l0, 16)] = vec
            return carry

        lax.fori_loop(0, hist, tr_idx, 0)

        def hc(g):
            h = g // cb_per_w
            cl = lax.rem(g, cb_per_w)
            return h, cl

        def gather_start(b, g):
            h, cl = hc(g)
            pltpu.make_async_copy(
                table_hbm.at[idx_t.at[h, cl]], rows_v.at[b], gsem.at[b]
            ).start()

        def gather_wait(b, g):
            h, cl = hc(g)
            pltpu.make_async_copy(
                table_hbm.at[idx_t.at[h, cl]], rows_v.at[b], gsem.at[b]
            ).wait()

        def writes_start(b, g):
            h, cl = hc(g)
            for d in range(dr):
                pltpu.make_async_copy(
                    tbuf.at[b, d], out_hbm.at[h, d, c0 + cl], wsem.at[b]
                ).start()

        def writes_wait(b, g):
            h, cl = hc(g)
            for d in range(dr):
                pltpu.make_async_copy(
                    tbuf.at[b, d], out_hbm.at[h, d, c0 + cl], wsem.at[b]
                ).wait()

        def transpose(b):
            rows = rows_v.at[b]
            for d in range(embed):
                col = jnp.full((16,), d, jnp.int32)
                for l0 in range(0, GROUP, 16):
                    vec = plsc.load_gather(rows, [l0 + lanes, col])
                    tbuf[b, d // 8, d % 8, pl.ds(l0, 16)] = vec

        gather_start(0, 0)
        gather_start(1, 1)

        def body(step, carry):
            g0 = step * NBUF
            for j in range(NBUF):
                g = g0 + j
                gather_wait(j, g)
                transpose(j)
                writes_start(j, g)
                b2 = (j + 2) % NBUF

                @pl.when(g >= 2)
                def _():
                    writes_wait(b2, g - 2)

                @pl.when(g + 2 < ngroups)
                def _():
                    gather_start(b2, g + 2)

            return carry

        lax.fori_loop(0, ngroups // NBUF, body, 0)

        for g in (ngroups - 2, ngroups - 1):
            writes_wait(g % NBUF, g)

    return k


def kernel(indices, table):
    batch, hist = indices.shape
    vocab, embed = table.shape
    out5 = _make_sc_gather(batch, hist, embed, vocab)(table, indices)
    return out5.transpose(2, 4, 0, 1, 3).reshape(batch, hist, embed)
